# Initial kernel scaffold; baseline (speedup 1.0000x reference)
import jax
import jax.numpy as jnp
from jax.experimental import pallas as pl


def kernel(pred, target):
    raise NotImplementedError("write your pallas kernel here")


# speedup vs baseline: 41.0911x; 41.0911x over previous
import functools

import jax
import jax.numpy as jnp
from jax import lax
from jax.experimental import pallas as pl
from jax.experimental.pallas import tpu as pltpu
from jax.experimental.pallas import tpu_sc as plsc

_BINS = 256
_ALPHA = 0.1
_NC = 2
_NS = 16
_NW = _NC * _NS
_LANES = 16

_N = 16 * 3 * 512 * 512
_N_PER_W = _N // _NW
_CHUNK = 32768
_NCH = _N_PER_W // _CHUNK


def _sc_body(pred_hbm, target_hbm, out_hbm, buf, hist, sem):
    wid = lax.axis_index("s") * _NC + lax.axis_index("c")

    def zero_step(i, carry):
        hist[pl.ds(i * _LANES, _LANES)] = jnp.zeros((_LANES,), jnp.float32)
        return carry

    lax.fori_loop(0, (_LANES * _BINS) // _LANES, zero_step, 0)

    lane_base = lax.iota(jnp.int32, _LANES) * _BINS
    base = wid * _N_PER_W

    def process(src_hbm, weight):
        wvec = jnp.full((_LANES,), weight, jnp.float32)

        def chunk_step(c, carry):
            pltpu.sync_copy(src_hbm.at[pl.ds(base + c * _CHUNK, _CHUNK)], buf)

            def step(i, inner):
                x = buf[pl.ds(i * _LANES, _LANES)]
                idx = (x * jnp.float32(_BINS)).astype(jnp.int32)
                idx = jnp.minimum(jnp.maximum(idx, 0), _BINS - 1)
                plsc.addupdate_scatter(hist, [idx + lane_base], wvec)
                return inner

            lax.fori_loop(0, _CHUNK // _LANES, step, 0)
            return carry

        lax.fori_loop(0, _NCH, chunk_step, 0)

    process(pred_hbm, 1.0)
    process(target_hbm, -1.0)
    pltpu.sync_copy(hist, out_hbm.at[wid])


def _tc_reduce_body(parts_ref, out_ref):
    s = jnp.sum(parts_ref[...], axis=0)
    loss = jnp.float32(_ALPHA) * jnp.sum(jnp.abs(s))
    out_ref[...] = loss.reshape(1, 1)


@jax.jit
def kernel(pred, target):
    pred_f = pred.reshape(-1)
    target_f = target.reshape(-1)

    mesh = plsc.VectorSubcoreMesh(
        core_axis_name="c", subcore_axis_name="s", num_cores=_NC,
        num_subcores=_NS,
    )
    parts = pl.kernel(
        _sc_body,
        out_type=jax.ShapeDtypeStruct((_NW, _LANES * _BINS), jnp.float32),
        mesh=mesh,
        scratch_types=[
            pltpu.VMEM((_CHUNK,), jnp.float32),
            pltpu.VMEM((_LANES * _BINS,), jnp.float32),
            pltpu.SemaphoreType.DMA,
        ],
        compiler_params=pltpu.CompilerParams(needs_layout_passes=False),
        name="cc_hist_sc",
    )(pred_f, target_f)

    loss = pl.pallas_call(
        _tc_reduce_body,
        out_shape=jax.ShapeDtypeStruct((1, 1), jnp.float32),
        name="cc_reduce_tc",
    )(parts.reshape(_NW * _LANES, _BINS))

    return loss[0, 0]


# speedup vs baseline: 133.3173x; 3.2444x over previous
import functools

import jax
import jax.numpy as jnp
from jax import lax
from jax.experimental import pallas as pl
from jax.experimental.pallas import tpu as pltpu
from jax.experimental.pallas import tpu_sc as plsc

_BINS = 256
_ALPHA = 0.1
_NC = 2
_NS = 16
_NW = _NC * _NS
_LANES = 16

_N = 16 * 3 * 512 * 512
_N_PER_W = _N // _NW
_CHUNK = 32768
_NCH = _N_PER_W // _CHUNK


def _sc_body(pred_hbm, target_hbm, out_hbm, buf, hist, sem0, sem1):
    wid = lax.axis_index("s") * _NC + lax.axis_index("c")

    def zero_step(i, carry):
        hist[pl.ds(i * _LANES, _LANES)] = jnp.zeros((_LANES,), jnp.float32)
        return carry

    lax.fori_loop(0, (_LANES * _BINS) // _LANES, zero_step, 0)

    lane_base = lax.iota(jnp.int32, _LANES) * _BINS
    base = wid * _N_PER_W
    sems = (sem0, sem1)

    sched = [(pred_hbm, c, 1.0) for c in range(_NCH)] + [
        (target_hbm, c, -1.0) for c in range(_NCH)
    ]

    def start(k):
        src, c, _ = sched[k]
        return pltpu.async_copy(
            src.at[pl.ds(base + c * _CHUNK, _CHUNK)],
            buf.at[k % 2],
            sems[k % 2],
        )

    def process_buf(b, weight):
        wvec = jnp.full((_LANES,), weight, jnp.float32)

        @plsc.parallel_loop(0, _CHUNK // _LANES, 1, unroll=8)
        def _(i):
            x = buf[b, pl.ds(i * _LANES, _LANES)]
            idx = jnp.minimum((x * jnp.float32(_BINS)).astype(jnp.int32),
                              _BINS - 1)
            plsc.addupdate_scatter(hist, [idx + lane_base], wvec)

    pending = start(0)
    for k in range(2 * _NCH):
        nxt = start(k + 1) if k + 1 < 2 * _NCH else None
        pending.wait()
        process_buf(k % 2, sched[k][2])
        pending = nxt
    pltpu.sync_copy(hist, out_hbm.at[wid])


def _tc_reduce_body(parts_ref, out_ref):
    s = jnp.sum(parts_ref[...], axis=0)
    loss = jnp.float32(_ALPHA) * jnp.sum(jnp.abs(s))
    out_ref[...] = loss.reshape(1, 1)


@jax.jit
def kernel(pred, target):
    pred_f = pred.reshape(-1)
    target_f = target.reshape(-1)

    mesh = plsc.VectorSubcoreMesh(
        core_axis_name="c", subcore_axis_name="s", num_cores=_NC,
        num_subcores=_NS,
    )
    parts = pl.kernel(
        _sc_body,
        out_type=jax.ShapeDtypeStruct((_NW, _LANES * _BINS), jnp.float32),
        mesh=mesh,
        scratch_types=[
            pltpu.VMEM((2, _CHUNK), jnp.float32),
            pltpu.VMEM((_LANES * _BINS,), jnp.float32),
            pltpu.SemaphoreType.DMA,
            pltpu.SemaphoreType.DMA,
        ],
        compiler_params=pltpu.CompilerParams(needs_layout_passes=False),
        name="cc_hist_sc",
    )(pred_f, target_f)

    loss = pl.pallas_call(
        _tc_reduce_body,
        out_shape=jax.ShapeDtypeStruct((1, 1), jnp.float32),
        name="cc_reduce_tc",
    )(parts.reshape(_NW * _LANES, _BINS))

    return loss[0, 0]


# speedup vs baseline: 149.0628x; 1.1181x over previous
import functools

import jax
import jax.numpy as jnp
from jax import lax
from jax.experimental import pallas as pl
from jax.experimental.pallas import tpu as pltpu
from jax.experimental.pallas import tpu_sc as plsc

_BINS = 256
_ALPHA = 0.1
_NC = 2
_NS = 16
_NW = _NC * _NS
_LANES = 16

_N = 16 * 3 * 512 * 512
_N_PER_W = _N // _NW
_CHUNK = 32768
_NCH = _N_PER_W // _CHUNK


def _sc_body(pred_hbm, target_hbm, out_hbm, buf, hist, out_small, sem0, sem1):
    wid = lax.axis_index("s") * _NC + lax.axis_index("c")

    def zero_step(i, carry):
        hist[pl.ds(i * _LANES, _LANES)] = jnp.zeros((_LANES,), jnp.float32)
        return carry

    lax.fori_loop(0, (_LANES * _BINS) // _LANES, zero_step, 0)

    lane_iota = lax.iota(jnp.int32, _LANES)
    base = wid * _N_PER_W
    sems = (sem0, sem1)

    sched = [(pred_hbm, c, 1.0) for c in range(_NCH)] + [
        (target_hbm, c, -1.0) for c in range(_NCH)
    ]

    def start(k):
        src, c, _ = sched[k]
        return pltpu.async_copy(
            src.at[pl.ds(base + c * _CHUNK, _CHUNK)],
            buf.at[k % 2],
            sems[k % 2],
        )

    def process_buf(b, weight):
        wvec = jnp.full((_LANES,), weight, jnp.float32)

        @plsc.parallel_loop(0, _CHUNK // _LANES, 1, unroll=8)
        def _(i):
            x = buf[b, pl.ds(i * _LANES, _LANES)]
            idx = jnp.minimum((x * jnp.float32(_BINS)).astype(jnp.int32),
                              _BINS - 1)
            plsc.addupdate_scatter(hist, [idx * _LANES + lane_iota], wvec)

    pending = start(0)
    for k in range(2 * _NCH):
        nxt = start(k + 1) if k + 1 < 2 * _NCH else None
        pending.wait()
        process_buf(k % 2, sched[k][2])
        pending = nxt

    for g in range(_BINS // _LANES):
        acc = jnp.zeros((_LANES,), jnp.float32)
        grp = (g * _LANES + lane_iota) * _LANES
        for l in range(_LANES):
            acc = acc + plsc.load_gather(hist, [grp + l])
        out_small[pl.ds(g * _LANES, _LANES)] = acc
    pltpu.sync_copy(out_small, out_hbm.at[wid])


def _tc_reduce_body(parts_ref, out_ref):
    s = jnp.sum(parts_ref[...], axis=0)
    loss = jnp.float32(_ALPHA) * jnp.sum(jnp.abs(s))
    out_ref[...] = loss.reshape(1, 1)


@jax.jit
def kernel(pred, target):
    pred_f = pred.reshape(-1)
    target_f = target.reshape(-1)

    mesh = plsc.VectorSubcoreMesh(
        core_axis_name="c", subcore_axis_name="s", num_cores=_NC,
        num_subcores=_NS,
    )
    parts = pl.kernel(
        _sc_body,
        out_type=jax.ShapeDtypeStruct((_NW, _BINS), jnp.float32),
        mesh=mesh,
        scratch_types=[
            pltpu.VMEM((2, _CHUNK), jnp.float32),
            pltpu.VMEM((_LANES * _BINS,), jnp.float32),
            pltpu.VMEM((_BINS,), jnp.float32),
            pltpu.SemaphoreType.DMA,
            pltpu.SemaphoreType.DMA,
        ],
        compiler_params=pltpu.CompilerParams(needs_layout_passes=False),
        name="cc_hist_sc",
    )(pred_f, target_f)

    loss = pl.pallas_call(
        _tc_reduce_body,
        out_shape=jax.ShapeDtypeStruct((1, 1), jnp.float32),
        name="cc_reduce_tc",
    )(parts)

    return loss[0, 0]
